# baseline (speedup 1.0000x reference)
import jax
import jax.numpy as jnp
from jax.experimental import pallas as pl
from jax.experimental.pallas import tpu as pltpu

N = 10000
IN_DIM = 128
HIDDEN = 256
LATENT = 64
AUG = HIDDEN + 16


def _hw1_kernel(x_ref, w_ref, b_ref, o_ref):
    r = (
        jnp.dot(x_ref[...], w_ref[...], preferred_element_type=jnp.float32)
        + b_ref[...]
    )
    ones = jnp.ones((r.shape[0], 1), jnp.float32)
    pad = jnp.zeros((r.shape[0], AUG - HIDDEN - 1), jnp.float32)
    o_ref[...] = jnp.concatenate([r, ones, pad], axis=1).astype(jnp.bfloat16)


def _hw1_aug(feats, w1, b1, block_rows=2000):
    return pl.pallas_call(
        _hw1_kernel,
        grid=(N // block_rows,),
        in_specs=[
            pl.BlockSpec((block_rows, IN_DIM), lambda i: (i, 0)),
            pl.BlockSpec((IN_DIM, HIDDEN), lambda i: (0, 0)),
            pl.BlockSpec((1, HIDDEN), lambda i: (0, 0)),
        ],
        out_specs=pl.BlockSpec((block_rows, AUG), lambda i: (i, 0)),
        out_shape=jax.ShapeDtypeStruct((N, AUG), jnp.bfloat16),
    )(feats, w1, b1.reshape(1, -1))


CHUNK = 1280
NPLANES = 8


def _pass1_kernel(a_ref, hw1_ref, w2_ref, b2_ref, m_ref, hw2_ref, s_ref):
    a = a_ref[...]
    m = (a > 0.0).astype(jnp.bfloat16)
    rows = m.shape[0]
    c_last = jnp.concatenate(
        [
            m[:, (NPLANES - 1) * CHUNK:],
            jnp.zeros((rows, NPLANES * CHUNK - N), jnp.bfloat16),
        ],
        axis=1,
    )
    packed = c_last * float(2 ** (NPLANES - 1))
    for k in range(NPLANES - 1):
        packed = packed + float(2 ** k) * m[:, k * CHUNK:(k + 1) * CHUNK]
    m_ref[...] = (packed - 128.0).astype(jnp.int8)
    r = jnp.dot(m, hw1_ref[...], preferred_element_type=jnp.float32)
    deg = r[:, HIDDEN:HIDDEN + 1]
    scale = 1.0 / jnp.maximum(deg, 1.0)
    s_ref[...] = scale
    h1 = jnp.maximum(r[:, :HIDDEN] * scale, 0.0)
    hw2 = (
        jnp.dot(
            h1.astype(jnp.bfloat16),
            w2_ref[...],
            preferred_element_type=jnp.float32,
        )
        + b2_ref[...]
    )
    hw2_ref[...] = hw2.astype(jnp.bfloat16)


def _pass1(a, hw1_aug, w2, b2, block_rows=400):
    return pl.pallas_call(
        _pass1_kernel,
        grid=(N // block_rows,),
        in_specs=[
            pl.BlockSpec((block_rows, N), lambda i: (i, 0)),
            pl.BlockSpec((N, AUG), lambda i: (0, 0)),
            pl.BlockSpec((HIDDEN, HIDDEN), lambda i: (0, 0)),
            pl.BlockSpec((1, HIDDEN), lambda i: (0, 0)),
        ],
        out_specs=[
            pl.BlockSpec((block_rows, CHUNK), lambda i: (i, 0)),
            pl.BlockSpec((block_rows, HIDDEN), lambda i: (i, 0)),
            pl.BlockSpec((block_rows, 1), lambda i: (i, 0)),
        ],
        out_shape=[
            jax.ShapeDtypeStruct((N, CHUNK), jnp.int8),
            jax.ShapeDtypeStruct((N, HIDDEN), jnp.bfloat16),
            jax.ShapeDtypeStruct((N, 1), jnp.float32),
        ],
        compiler_params=pltpu.CompilerParams(
            dimension_semantics=("parallel",),
        ),
    )(a, hw1_aug, w2, b2.reshape(1, -1))


def _pass2_kernel(m_ref, hw2_ref, s_ref, wh_ref, bh_ref, mu_ref, lv_ref):
    p = m_ref[...].astype(jnp.int32) + 128
    r = jnp.zeros((p.shape[0], HIDDEN), jnp.float32)
    for k in range(NPLANES):
        bit = jnp.bitwise_and(jnp.right_shift(p, k), 1).astype(jnp.bfloat16)
        r = r + jnp.dot(
            bit,
            hw2_ref[k * CHUNK:(k + 1) * CHUNK, :],
            preferred_element_type=jnp.float32,
        )
    h2 = jnp.maximum(r * s_ref[...], 0.0)
    out = (
        jnp.dot(
            h2.astype(jnp.bfloat16),
            wh_ref[...],
            preferred_element_type=jnp.float32,
        )
        + bh_ref[...]
    )
    mu_ref[...] = out[:, :LATENT]
    lv_ref[...] = out[:, LATENT:]


def _pass2(m_i8, hw2_bf16, scale, w_heads, b_heads, block_rows=1000):
    return pl.pallas_call(
        _pass2_kernel,
        grid=(N // block_rows,),
        in_specs=[
            pl.BlockSpec((block_rows, CHUNK), lambda i: (i, 0)),
            pl.BlockSpec((NPLANES * CHUNK, HIDDEN), lambda i: (0, 0)),
            pl.BlockSpec((block_rows, 1), lambda i: (i, 0)),
            pl.BlockSpec((HIDDEN, 2 * LATENT), lambda i: (0, 0)),
            pl.BlockSpec((1, 2 * LATENT), lambda i: (0, 0)),
        ],
        out_specs=[
            pl.BlockSpec((block_rows, LATENT), lambda i: (i, 0)),
            pl.BlockSpec((block_rows, LATENT), lambda i: (i, 0)),
        ],
        out_shape=[
            jax.ShapeDtypeStruct((N, LATENT), jnp.float32),
            jax.ShapeDtypeStruct((N, LATENT), jnp.float32),
        ],
        compiler_params=pltpu.CompilerParams(
            dimension_semantics=("parallel",),
        ),
    )(m_i8, hw2_bf16, scale, w_heads, b_heads)


def kernel(A_norm, feats, W1, b1, W2, b2, Wmu, bmu, Wlv, blv):
    hw1_aug = _hw1_aug(feats, W1, b1)
    m_i8, hw2, scale = _pass1(A_norm, hw1_aug, W2, b2)
    hw2_pad = jnp.concatenate(
        [hw2, jnp.zeros((NPLANES * CHUNK - N, HIDDEN), jnp.bfloat16)], axis=0
    )
    w_heads = jnp.concatenate([Wmu, Wlv], axis=1).astype(jnp.bfloat16)
    b_heads = jnp.concatenate([bmu, blv], axis=0).reshape(1, -1)
    mu, logvar = _pass2(m_i8, hw2_pad, scale, w_heads, b_heads)
    return (mu, logvar)


# speedup vs baseline: 1.0217x; 1.0217x over previous
import jax
import jax.numpy as jnp
from jax.experimental import pallas as pl
from jax.experimental.pallas import tpu as pltpu

N = 10000
IN_DIM = 128
HIDDEN = 256
LATENT = 64
AUG = HIDDEN + 16


def _hw1_kernel(x_ref, w_ref, b_ref, o_ref):
    r = (
        jnp.dot(x_ref[...], w_ref[...], preferred_element_type=jnp.float32)
        + b_ref[...]
    )
    ones = jnp.ones((r.shape[0], 1), jnp.float32)
    pad = jnp.zeros((r.shape[0], AUG - HIDDEN - 1), jnp.float32)
    o_ref[...] = jnp.concatenate([r, ones, pad], axis=1).astype(jnp.bfloat16)


def _hw1_aug(feats, w1, b1, block_rows=2000):
    return pl.pallas_call(
        _hw1_kernel,
        grid=(N // block_rows,),
        in_specs=[
            pl.BlockSpec((block_rows, IN_DIM), lambda i: (i, 0)),
            pl.BlockSpec((IN_DIM, HIDDEN), lambda i: (0, 0)),
            pl.BlockSpec((1, HIDDEN), lambda i: (0, 0)),
        ],
        out_specs=pl.BlockSpec((block_rows, AUG), lambda i: (i, 0)),
        out_shape=jax.ShapeDtypeStruct((N, AUG), jnp.bfloat16),
    )(feats, w1, b1.reshape(1, -1))


CHUNK = 2560


def _pass1_kernel(a_ref, hw1_ref, w2_ref, b2_ref, m_ref, hw2_ref, s_ref):
    a = a_ref[...]
    m = (a > 0.0).astype(jnp.bfloat16)
    rows = m.shape[0]
    c3 = jnp.concatenate(
        [m[:, 3 * CHUNK:], jnp.zeros((rows, 4 * CHUNK - N), jnp.bfloat16)],
        axis=1,
    )
    packed = (
        m[:, :CHUNK]
        + 2.0 * m[:, CHUNK:2 * CHUNK]
        + 4.0 * m[:, 2 * CHUNK:3 * CHUNK]
        + 8.0 * c3
    )
    m_ref[...] = packed.astype(jnp.int8)
    r = jnp.dot(m, hw1_ref[...], preferred_element_type=jnp.float32)
    deg = r[:, HIDDEN:HIDDEN + 1]
    scale = 1.0 / jnp.maximum(deg, 1.0)
    s_ref[...] = scale
    h1 = jnp.maximum(r[:, :HIDDEN] * scale, 0.0)
    hw2 = (
        jnp.dot(
            h1.astype(jnp.bfloat16),
            w2_ref[...],
            preferred_element_type=jnp.float32,
        )
        + b2_ref[...]
    )
    hw2_ref[...] = hw2.astype(jnp.bfloat16)


def _pass1(a, hw1_aug, w2, b2, block_rows=400):
    return pl.pallas_call(
        _pass1_kernel,
        grid=(N // block_rows,),
        in_specs=[
            pl.BlockSpec((block_rows, N), lambda i: (i, 0)),
            pl.BlockSpec((N, AUG), lambda i: (0, 0)),
            pl.BlockSpec((HIDDEN, HIDDEN), lambda i: (0, 0)),
            pl.BlockSpec((1, HIDDEN), lambda i: (0, 0)),
        ],
        out_specs=[
            pl.BlockSpec((block_rows, CHUNK), lambda i: (i, 0)),
            pl.BlockSpec((block_rows, HIDDEN), lambda i: (i, 0)),
            pl.BlockSpec((block_rows, 1), lambda i: (i, 0)),
        ],
        out_shape=[
            jax.ShapeDtypeStruct((N, CHUNK), jnp.int8),
            jax.ShapeDtypeStruct((N, HIDDEN), jnp.bfloat16),
            jax.ShapeDtypeStruct((N, 1), jnp.float32),
        ],
        compiler_params=pltpu.CompilerParams(
            dimension_semantics=("parallel",),
        ),
    )(a, hw1_aug, w2, b2.reshape(1, -1))


def _pass2_kernel(m_ref, hw2_ref, s_ref, wh_ref, bh_ref, mu_ref, lv_ref):
    p = m_ref[...].astype(jnp.int32)
    r = jnp.zeros((p.shape[0], HIDDEN), jnp.float32)
    for k in range(4):
        bit = jnp.bitwise_and(jnp.right_shift(p, k), 1).astype(jnp.bfloat16)
        r = r + jnp.dot(
            bit,
            hw2_ref[k * CHUNK:(k + 1) * CHUNK, :],
            preferred_element_type=jnp.float32,
        )
    h2 = jnp.maximum(r * s_ref[...], 0.0)
    out = (
        jnp.dot(
            h2.astype(jnp.bfloat16),
            wh_ref[...],
            preferred_element_type=jnp.float32,
        )
        + bh_ref[...]
    )
    mu_ref[...] = out[:, :LATENT]
    lv_ref[...] = out[:, LATENT:]


def _pass2(m_i8, hw2_bf16, scale, w_heads, b_heads, block_rows=1000):
    return pl.pallas_call(
        _pass2_kernel,
        grid=(N // block_rows,),
        in_specs=[
            pl.BlockSpec((block_rows, CHUNK), lambda i: (i, 0)),
            pl.BlockSpec((4 * CHUNK, HIDDEN), lambda i: (0, 0)),
            pl.BlockSpec((block_rows, 1), lambda i: (i, 0)),
            pl.BlockSpec((HIDDEN, 2 * LATENT), lambda i: (0, 0)),
            pl.BlockSpec((1, 2 * LATENT), lambda i: (0, 0)),
        ],
        out_specs=[
            pl.BlockSpec((block_rows, LATENT), lambda i: (i, 0)),
            pl.BlockSpec((block_rows, LATENT), lambda i: (i, 0)),
        ],
        out_shape=[
            jax.ShapeDtypeStruct((N, LATENT), jnp.float32),
            jax.ShapeDtypeStruct((N, LATENT), jnp.float32),
        ],
        compiler_params=pltpu.CompilerParams(
            dimension_semantics=("parallel",),
        ),
    )(m_i8, hw2_bf16, scale, w_heads, b_heads)


def kernel(A_norm, feats, W1, b1, W2, b2, Wmu, bmu, Wlv, blv):
    hw1_aug = _hw1_aug(feats, W1, b1)
    m_i8, hw2, scale = _pass1(A_norm, hw1_aug, W2, b2)
    hw2_pad = jnp.concatenate(
        [hw2, jnp.zeros((4 * CHUNK - N, HIDDEN), jnp.bfloat16)], axis=0
    )
    w_heads = jnp.concatenate([Wmu, Wlv], axis=1).astype(jnp.bfloat16)
    b_heads = jnp.concatenate([bmu, blv], axis=0).reshape(1, -1)
    mu, logvar = _pass2(m_i8, hw2_pad, scale, w_heads, b_heads)
    return (mu, logvar)
